# Initial kernel scaffold; baseline (speedup 1.0000x reference)
import jax
import jax.numpy as jnp
from jax.experimental import pallas as pl


def kernel(x, edge_index, W1, b1, W2, b2):
    raise NotImplementedError("write your pallas kernel here")


# speedup vs baseline: 14.0368x; 14.0368x over previous
import functools

import jax
import jax.numpy as jnp
from jax import lax
from jax.experimental import pallas as pl
from jax.experimental.pallas import tpu as pltpu
from jax.experimental.pallas import tpu_sc as plsc

N = 10000
E = 160000
NS = 16
NC = 2
WRITERS = 10
ROWS_PER_WRITER = N // WRITERS
EDGES_PER_TILE = E // NS
CHUNK = 125
NCHUNK = EDGES_PER_TILE // CHUNK
EDGES_PER_WORKER = E // (NC * NS)
DCHUNK = 125
DNCHUNK = EDGES_PER_WORKER // DCHUNK
RB = 1000
NRB = N // RB

_MESH = plsc.VectorSubcoreMesh(core_axis_name="c", subcore_axis_name="s")


def _sc_degree(dst_w, zeros_nf, ones_blk, W):
    @functools.partial(
        pl.kernel,
        out_type=jax.ShapeDtypeStruct((NC, N, W), jnp.float32),
        mesh=_MESH,
        scratch_types=[
            pltpu.VMEM((DNCHUNK, DCHUNK), jnp.int32),
            pltpu.VMEM((DCHUNK, W), jnp.float32),
            pltpu.VMEM_SHARED((N, W), jnp.float32),
        ],
    )
    def k(dst_hbm, z_hbm, ones_hbm, out_hbm, dbuf, ones_v, accd):
        c = lax.axis_index("c")
        s = lax.axis_index("s")
        pltpu.sync_copy(dst_hbm.at[c, s], dbuf)
        pltpu.sync_copy(ones_hbm, ones_v)
        r0 = s * ROWS_PER_WRITER

        @pl.when(s < WRITERS)
        def _():
            pltpu.sync_copy(z_hbm.at[pl.ds(r0, ROWS_PER_WRITER)],
                            accd.at[pl.ds(r0, ROWS_PER_WRITER)])

        plsc.subcore_barrier()

        @pl.loop(0, DNCHUNK)
        def _(j):
            pltpu.sync_copy(ones_v, accd.at[dbuf.at[j]], add=True)

        plsc.subcore_barrier()

        @pl.when(s < WRITERS)
        def _():
            pltpu.sync_copy(accd.at[pl.ds(r0, ROWS_PER_WRITER)],
                            out_hbm.at[c, pl.ds(r0, ROWS_PER_WRITER)])

    return k(dst_w, zeros_nf, ones_blk)


def _sc_spmm(g_flat, src_all, dst_t, zeros_nf, F):
    @functools.partial(
        pl.kernel,
        out_type=jax.ShapeDtypeStruct((NC, N, F), jnp.float32),
        mesh=_MESH,
        scratch_types=[
            pltpu.VMEM((NCHUNK, CHUNK), jnp.int32),
            pltpu.VMEM((NCHUNK, CHUNK), jnp.int32),
            pltpu.VMEM((CHUNK, F), jnp.float32),
            pltpu.VMEM_SHARED((N, F), jnp.float32),
        ],
    )
    def k(g_hbm, src_hbm, dst_hbm, z_hbm, out_hbm, sbuf, dbuf, mb, acc):
        c = lax.axis_index("c")
        s = lax.axis_index("s")
        pltpu.sync_copy(src_hbm.at[c, s], sbuf)
        pltpu.sync_copy(dst_hbm.at[s], dbuf)
        r0 = s * ROWS_PER_WRITER

        @pl.when(s < WRITERS)
        def _():
            pltpu.sync_copy(z_hbm.at[pl.ds(r0, ROWS_PER_WRITER)],
                            acc.at[pl.ds(r0, ROWS_PER_WRITER)])

        plsc.subcore_barrier()

        @pl.loop(0, NCHUNK)
        def _(j):
            pltpu.sync_copy(g_hbm.at[sbuf.at[j]], mb)
            pltpu.sync_copy(mb, acc.at[dbuf.at[j]], add=True)

        plsc.subcore_barrier()

        @pl.when(s < WRITERS)
        def _():
            pltpu.sync_copy(acc.at[pl.ds(r0, ROWS_PER_WRITER)],
                            out_hbm.at[c, pl.ds(r0, ROWS_PER_WRITER)])

    return k(g_flat, src_all, dst_t, zeros_nf)


def _sc_spmm_edgesplit(g, src_w, dst_w2, zeros_nf, F):
    @functools.partial(
        pl.kernel,
        out_type=jax.ShapeDtypeStruct((NC, N, F), jnp.float32),
        mesh=_MESH,
        scratch_types=[
            pltpu.VMEM((DNCHUNK, DCHUNK), jnp.int32),
            pltpu.VMEM((DNCHUNK, DCHUNK), jnp.int32),
            pltpu.VMEM((DCHUNK, F), jnp.float32),
            pltpu.VMEM_SHARED((N, F), jnp.float32),
        ],
    )
    def k(g_hbm, src_hbm, dst_hbm, z_hbm, out_hbm, sbuf, dbuf, mb, acc):
        c = lax.axis_index("c")
        s = lax.axis_index("s")
        pltpu.sync_copy(src_hbm.at[c, s], sbuf)
        pltpu.sync_copy(dst_hbm.at[c, s], dbuf)
        r0 = s * ROWS_PER_WRITER

        @pl.when(s < WRITERS)
        def _():
            pltpu.sync_copy(z_hbm.at[pl.ds(r0, ROWS_PER_WRITER)],
                            acc.at[pl.ds(r0, ROWS_PER_WRITER)])

        plsc.subcore_barrier()

        @pl.loop(0, DNCHUNK)
        def _(j):
            pltpu.sync_copy(g_hbm.at[sbuf.at[j]], mb)
            pltpu.sync_copy(mb, acc.at[dbuf.at[j]], add=True)

        plsc.subcore_barrier()

        @pl.when(s < WRITERS)
        def _():
            pltpu.sync_copy(acc.at[pl.ds(r0, ROWS_PER_WRITER)],
                            out_hbm.at[c, pl.ds(r0, ROWS_PER_WRITER)])

    return k(g, src_w, dst_w2, zeros_nf)


def _tc_mm1(x, W1):
    D = W1.shape[0]

    def body(x_ref, w_ref, o_ref):
        o_ref[...] = jnp.dot(x_ref[...], w_ref[...],
                             preferred_element_type=jnp.float32)

    return pl.pallas_call(
        body,
        grid=(NRB,),
        in_specs=[pl.BlockSpec((RB, D), lambda i: (i, 0)),
                  pl.BlockSpec((D, D), lambda i: (0, 0))],
        out_specs=pl.BlockSpec((RB, D), lambda i: (i, 0)),
        out_shape=jax.ShapeDtypeStruct((N, D), jnp.float32),
    )(x, W1)


def _tc_scale1(h1, degparts):
    D = h1.shape[1]
    H = D // 2

    def body(h_ref, d_ref, g_ref, dinv_ref):
        deg = d_ref[0, 0, :, 0:1] + d_ref[1, 0, :, 0:1] + 1.0
        dinv = lax.rsqrt(deg)
        dinv_ref[0] = dinv
        g = h_ref[...] * dinv
        g_ref[0] = g[:, :H]
        g_ref[1] = g[:, H:]

    return pl.pallas_call(
        body,
        grid=(NRB,),
        in_specs=[pl.BlockSpec((RB, D), lambda i: (i, 0)),
                  pl.BlockSpec((NC, 1, RB, H), lambda i: (0, i, 0, 0))],
        out_specs=[pl.BlockSpec((NC, RB, H), lambda i: (0, i, 0)),
                   pl.BlockSpec((1, RB, 1), lambda i: (i, 0, 0))],
        out_shape=[jax.ShapeDtypeStruct((NC, N, H), jnp.float32),
                   jax.ShapeDtypeStruct((NRB, RB, 1), jnp.float32)],
    )(h1, degparts)


def _tc_mid(acc1, g1, dinv, b1, W2):
    H = acc1.shape[2]
    D = 2 * H
    DO = W2.shape[1]

    def body(a_ref, g_ref, d_ref, b_ref, w_ref, o_ref):
        dv = d_ref[0]
        full = jnp.concatenate([a_ref[0] + g_ref[0], a_ref[1] + g_ref[1]],
                               axis=1)
        h = jnp.maximum(full * dv + b_ref[...], 0.0)
        hw = jnp.dot(h, w_ref[...], preferred_element_type=jnp.float32)
        o_ref[...] = hw * dv

    return pl.pallas_call(
        body,
        grid=(NRB,),
        in_specs=[pl.BlockSpec((NC, RB, H), lambda i: (0, i, 0)),
                  pl.BlockSpec((NC, RB, H), lambda i: (0, i, 0)),
                  pl.BlockSpec((1, RB, 1), lambda i: (i, 0, 0)),
                  pl.BlockSpec((1, D), lambda i: (0, 0)),
                  pl.BlockSpec((D, DO), lambda i: (0, 0))],
        out_specs=pl.BlockSpec((RB, DO), lambda i: (i, 0)),
        out_shape=jax.ShapeDtypeStruct((N, DO), jnp.float32),
    )(acc1, g1, dinv, b1, W2)


def _tc_final(acc2, g2, dinv, b2):
    DO = g2.shape[1]

    def body(a_ref, g_ref, d_ref, b_ref, o_ref):
        dv = d_ref[0]
        o = (a_ref[0] + a_ref[1] + g_ref[...]) * dv + b_ref[...]
        m = jnp.max(o, axis=1, keepdims=True)
        e = jnp.exp(o - m)
        o_ref[...] = (o - m) - jnp.log(jnp.sum(e, axis=1, keepdims=True))

    return pl.pallas_call(
        body,
        grid=(NRB,),
        in_specs=[pl.BlockSpec((NC, RB, DO), lambda i: (0, i, 0)),
                  pl.BlockSpec((RB, DO), lambda i: (i, 0)),
                  pl.BlockSpec((1, RB, 1), lambda i: (i, 0, 0)),
                  pl.BlockSpec((1, DO), lambda i: (0, 0))],
        out_specs=pl.BlockSpec((RB, DO), lambda i: (i, 0)),
        out_shape=jax.ShapeDtypeStruct((N, DO), jnp.float32),
    )(acc2, g2, dinv, b2)


def kernel(x, edge_index, W1, b1, W2, b2):
    D = W1.shape[0]
    H = D // 2
    DO = W2.shape[1]
    src = edge_index[0]
    dst = edge_index[1]

    srcr = src.reshape(NS, NCHUNK, CHUNK)
    src_all = srcr[None] + (jnp.arange(NC, dtype=jnp.int32) * N)[:, None, None, None]
    dst_t = dst.reshape(NS, NCHUNK, CHUNK)
    src_w = src.reshape(NC, NS, DNCHUNK, DCHUNK)
    dst_w = dst.reshape(NC, NS, DNCHUNK, DCHUNK)
    ones_blk = jnp.ones((DCHUNK, H), jnp.float32)
    zeros_h = jnp.zeros((N, H), jnp.float32)
    zeros_do = jnp.zeros((N, DO), jnp.float32)

    degparts = _sc_degree(dst_w, zeros_h, ones_blk, H)
    h1 = _tc_mm1(x, W1)
    g1, dinv = _tc_scale1(h1, degparts.reshape(NC, NRB, RB, H))
    acc1 = _sc_spmm(g1.reshape(NC * N, H), src_all, dst_t, zeros_h, H)
    g2 = _tc_mid(acc1, g1, dinv, b1.reshape(1, D), W2)
    acc2 = _sc_spmm_edgesplit(g2, src_w, dst_w, zeros_do, DO)
    return _tc_final(acc2, g2, dinv, b2.reshape(1, DO))


# speedup vs baseline: 16.1934x; 1.1536x over previous
import functools

import jax
import jax.numpy as jnp
from jax import lax
from jax.experimental import pallas as pl
from jax.experimental.pallas import tpu as pltpu
from jax.experimental.pallas import tpu_sc as plsc

N = 10000
E = 160000
NS = 16
NC = 2
WRITERS = 10
ROWS_PER_WRITER = N // WRITERS
EDGES_PER_TILE = E // NS
CHUNK = 100
CHUNKP = 104
NCHUNK = EDGES_PER_TILE // CHUNK
EDGES_PER_WORKER = E // (NC * NS)
CHUNK2 = 100
CHUNKP2 = 104
NCHUNK2 = EDGES_PER_WORKER // CHUNK2
DCHUNK = 125
DNCHUNK = EDGES_PER_WORKER // DCHUNK
RB = 1000
NRB = N // RB

_MESH = plsc.VectorSubcoreMesh(core_axis_name="c", subcore_axis_name="s")


def _sc_degree(dst_w, zeros_nf, ones_blk, W):
    @functools.partial(
        pl.kernel,
        out_type=jax.ShapeDtypeStruct((NC, N, W), jnp.float32),
        mesh=_MESH,
        scratch_types=[
            pltpu.VMEM((DNCHUNK, DCHUNK), jnp.int32),
            pltpu.VMEM((DCHUNK, W), jnp.float32),
            pltpu.VMEM_SHARED((N, W), jnp.float32),
            pltpu.SemaphoreType.DMA,
        ],
    )
    def k(dst_hbm, z_hbm, ones_hbm, out_hbm, dbuf, ones_v, accd, sem):
        c = lax.axis_index("c")
        s = lax.axis_index("s")
        pltpu.sync_copy(dst_hbm.at[c, s], dbuf)
        pltpu.sync_copy(ones_hbm, ones_v)
        r0 = s * ROWS_PER_WRITER

        @pl.when(s < WRITERS)
        def _():
            pltpu.sync_copy(z_hbm.at[pl.ds(r0, ROWS_PER_WRITER)],
                            accd.at[pl.ds(r0, ROWS_PER_WRITER)])

        plsc.subcore_barrier()

        @pl.loop(0, DNCHUNK)
        def _(j):
            pltpu.async_copy(ones_v, accd.at[dbuf.at[j]], sem, add=True)

        @pl.loop(0, DNCHUNK)
        def _(j):
            pltpu.make_async_copy(ones_v, accd.at[dbuf.at[j]], sem).wait()

        plsc.subcore_barrier()

        @pl.when(s < WRITERS)
        def _():
            pltpu.sync_copy(accd.at[pl.ds(r0, ROWS_PER_WRITER)],
                            out_hbm.at[c, pl.ds(r0, ROWS_PER_WRITER)])

    return k(dst_w, zeros_nf, ones_blk)


def _sc_spmm(g_flat, src_all, dst_t, zeros_nf, F):
    @functools.partial(
        pl.kernel,
        out_type=jax.ShapeDtypeStruct((NC, N, F), jnp.float32),
        mesh=_MESH,
        scratch_types=[
            pltpu.VMEM((NCHUNK * CHUNKP,), jnp.int32),
            pltpu.VMEM((NCHUNK, CHUNK), jnp.int32),
            pltpu.VMEM((CHUNK, F), jnp.float32),
            pltpu.VMEM((CHUNK, F), jnp.float32),
            pltpu.VMEM_SHARED((N, F), jnp.float32),
            pltpu.SemaphoreType.DMA,
            pltpu.SemaphoreType.DMA,
        ],
    )
    def k(g_hbm, src_hbm, dst_hbm, z_hbm, out_hbm,
          sbuf, dbuf, mb0, mb1, acc, sem0, sem1):
        c = lax.axis_index("c")
        s = lax.axis_index("s")
        pltpu.sync_copy(src_hbm.at[c, s], sbuf)
        pltpu.sync_copy(dst_hbm.at[s], dbuf)
        r0 = s * ROWS_PER_WRITER

        @pl.when(s < WRITERS)
        def _():
            pltpu.sync_copy(z_hbm.at[pl.ds(r0, ROWS_PER_WRITER)],
                            acc.at[pl.ds(r0, ROWS_PER_WRITER)])

        plsc.subcore_barrier()

        def sidx(j):
            return sbuf.at[pl.ds(pl.multiple_of(j * CHUNKP, 8), CHUNK)]

        pltpu.async_copy(g_hbm.at[sidx(0)], mb0, sem0)

        @pl.loop(0, NCHUNK, step=2)
        def _(j):
            pltpu.make_async_copy(g_hbm.at[sidx(j)], mb0, sem0).wait()
            pltpu.async_copy(g_hbm.at[sidx(j + 1)], mb1, sem1)
            pltpu.sync_copy(mb0, acc.at[dbuf.at[j]], add=True)
            pltpu.make_async_copy(g_hbm.at[sidx(j + 1)], mb1, sem1).wait()

            @pl.when(j + 2 < NCHUNK)
            def _():
                pltpu.async_copy(g_hbm.at[sidx(j + 2)], mb0, sem0)

            pltpu.sync_copy(mb1, acc.at[dbuf.at[j + 1]], add=True)

        plsc.subcore_barrier()

        @pl.when(s < WRITERS)
        def _():
            pltpu.sync_copy(acc.at[pl.ds(r0, ROWS_PER_WRITER)],
                            out_hbm.at[c, pl.ds(r0, ROWS_PER_WRITER)])

    return k(g_flat, src_all, dst_t, zeros_nf)


def _sc_spmm_edgesplit(g, src_w, dst_w2, zeros_nf, F):
    @functools.partial(
        pl.kernel,
        out_type=jax.ShapeDtypeStruct((NC, N, F), jnp.float32),
        mesh=_MESH,
        scratch_types=[
            pltpu.VMEM((NCHUNK2 * CHUNKP2,), jnp.int32),
            pltpu.VMEM((NCHUNK2, CHUNK2), jnp.int32),
            pltpu.VMEM((CHUNK2, F), jnp.float32),
            pltpu.VMEM((CHUNK2, F), jnp.float32),
            pltpu.VMEM_SHARED((N, F), jnp.float32),
            pltpu.SemaphoreType.DMA,
            pltpu.SemaphoreType.DMA,
        ],
    )
    def k(g_hbm, src_hbm, dst_hbm, z_hbm, out_hbm,
          sbuf, dbuf, mb0, mb1, acc, sem0, sem1):
        c = lax.axis_index("c")
        s = lax.axis_index("s")
        pltpu.sync_copy(src_hbm.at[c, s], sbuf)
        pltpu.sync_copy(dst_hbm.at[c, s], dbuf)
        r0 = s * ROWS_PER_WRITER

        @pl.when(s < WRITERS)
        def _():
            pltpu.sync_copy(z_hbm.at[pl.ds(r0, ROWS_PER_WRITER)],
                            acc.at[pl.ds(r0, ROWS_PER_WRITER)])

        plsc.subcore_barrier()

        def sidx(j):
            return sbuf.at[pl.ds(pl.multiple_of(j * CHUNKP2, 8), CHUNK2)]

        pltpu.async_copy(g_hbm.at[sidx(0)], mb0, sem0)

        @pl.loop(0, NCHUNK2, step=2)
        def _(j):
            pltpu.make_async_copy(g_hbm.at[sidx(j)], mb0, sem0).wait()
            pltpu.async_copy(g_hbm.at[sidx(j + 1)], mb1, sem1)
            pltpu.sync_copy(mb0, acc.at[dbuf.at[j]], add=True)
            pltpu.make_async_copy(g_hbm.at[sidx(j + 1)], mb1, sem1).wait()

            @pl.when(j + 2 < NCHUNK2)
            def _():
                pltpu.async_copy(g_hbm.at[sidx(j + 2)], mb0, sem0)

            pltpu.sync_copy(mb1, acc.at[dbuf.at[j + 1]], add=True)

        plsc.subcore_barrier()

        @pl.when(s < WRITERS)
        def _():
            pltpu.sync_copy(acc.at[pl.ds(r0, ROWS_PER_WRITER)],
                            out_hbm.at[c, pl.ds(r0, ROWS_PER_WRITER)])

    return k(g, src_w, dst_w2, zeros_nf)


def _tc_mm1(x, W1):
    D = W1.shape[0]

    def body(x_ref, w_ref, o_ref):
        o_ref[...] = jnp.dot(x_ref[...], w_ref[...],
                             preferred_element_type=jnp.float32)

    return pl.pallas_call(
        body,
        grid=(NRB,),
        in_specs=[pl.BlockSpec((RB, D), lambda i: (i, 0)),
                  pl.BlockSpec((D, D), lambda i: (0, 0))],
        out_specs=pl.BlockSpec((RB, D), lambda i: (i, 0)),
        out_shape=jax.ShapeDtypeStruct((N, D), jnp.float32),
    )(x, W1)


def _tc_scale1(h1, degparts):
    D = h1.shape[1]
    H = D // 2

    def body(h_ref, d_ref, g_ref, dinv_ref):
        deg = d_ref[0, 0, :, 0:1] + d_ref[1, 0, :, 0:1] + 1.0
        dinv = lax.rsqrt(deg)
        dinv_ref[0] = dinv
        g = h_ref[...] * dinv
        g_ref[0] = g[:, :H]
        g_ref[1] = g[:, H:]

    return pl.pallas_call(
        body,
        grid=(NRB,),
        in_specs=[pl.BlockSpec((RB, D), lambda i: (i, 0)),
                  pl.BlockSpec((NC, 1, RB, H), lambda i: (0, i, 0, 0))],
        out_specs=[pl.BlockSpec((NC, RB, H), lambda i: (0, i, 0)),
                   pl.BlockSpec((1, RB, 1), lambda i: (i, 0, 0))],
        out_shape=[jax.ShapeDtypeStruct((NC, N, H), jnp.float32),
                   jax.ShapeDtypeStruct((NRB, RB, 1), jnp.float32)],
    )(h1, degparts)


def _tc_mid(acc1, g1, dinv, b1, W2):
    H = acc1.shape[2]
    D = 2 * H
    DO = W2.shape[1]

    def body(a_ref, g_ref, d_ref, b_ref, w_ref, o_ref):
        dv = d_ref[0]
        full = jnp.concatenate([a_ref[0] + g_ref[0], a_ref[1] + g_ref[1]],
                               axis=1)
        h = jnp.maximum(full * dv + b_ref[...], 0.0)
        hw = jnp.dot(h, w_ref[...], preferred_element_type=jnp.float32)
        o_ref[...] = hw * dv

    return pl.pallas_call(
        body,
        grid=(NRB,),
        in_specs=[pl.BlockSpec((NC, RB, H), lambda i: (0, i, 0)),
                  pl.BlockSpec((NC, RB, H), lambda i: (0, i, 0)),
                  pl.BlockSpec((1, RB, 1), lambda i: (i, 0, 0)),
                  pl.BlockSpec((1, D), lambda i: (0, 0)),
                  pl.BlockSpec((D, DO), lambda i: (0, 0))],
        out_specs=pl.BlockSpec((RB, DO), lambda i: (i, 0)),
        out_shape=jax.ShapeDtypeStruct((N, DO), jnp.float32),
    )(acc1, g1, dinv, b1, W2)


def _tc_final(acc2, g2, dinv, b2):
    DO = g2.shape[1]

    def body(a_ref, g_ref, d_ref, b_ref, o_ref):
        dv = d_ref[0]
        o = (a_ref[0] + a_ref[1] + g_ref[...]) * dv + b_ref[...]
        m = jnp.max(o, axis=1, keepdims=True)
        e = jnp.exp(o - m)
        o_ref[...] = (o - m) - jnp.log(jnp.sum(e, axis=1, keepdims=True))

    return pl.pallas_call(
        body,
        grid=(NRB,),
        in_specs=[pl.BlockSpec((NC, RB, DO), lambda i: (0, i, 0)),
                  pl.BlockSpec((RB, DO), lambda i: (i, 0)),
                  pl.BlockSpec((1, RB, 1), lambda i: (i, 0, 0)),
                  pl.BlockSpec((1, DO), lambda i: (0, 0))],
        out_specs=pl.BlockSpec((RB, DO), lambda i: (i, 0)),
        out_shape=jax.ShapeDtypeStruct((N, DO), jnp.float32),
    )(acc2, g2, dinv, b2)


def kernel(x, edge_index, W1, b1, W2, b2):
    D = W1.shape[0]
    H = D // 2
    DO = W2.shape[1]
    src = edge_index[0]
    dst = edge_index[1]

    srcr = jnp.pad(src.reshape(NS, NCHUNK, CHUNK),
                   ((0, 0), (0, 0), (0, CHUNKP - CHUNK)))
    src_all = (srcr[None] + (jnp.arange(NC, dtype=jnp.int32) * N)[:, None, None, None]
               ).reshape(NC, NS, NCHUNK * CHUNKP)
    dst_t = dst.reshape(NS, NCHUNK, CHUNK)
    src_w = jnp.pad(src.reshape(NC, NS, NCHUNK2, CHUNK2),
                    ((0, 0), (0, 0), (0, 0), (0, CHUNKP2 - CHUNK2))
                    ).reshape(NC, NS, NCHUNK2 * CHUNKP2)
    dst_w2 = dst.reshape(NC, NS, NCHUNK2, CHUNK2)
    dst_w = dst.reshape(NC, NS, DNCHUNK, DCHUNK)
    ones_blk = jnp.ones((DCHUNK, H), jnp.float32)
    zeros_h = jnp.zeros((N, H), jnp.float32)
    zeros_do = jnp.zeros((N, DO), jnp.float32)

    degparts = _sc_degree(dst_w, zeros_h, ones_blk, H)
    h1 = _tc_mm1(x, W1)
    g1, dinv = _tc_scale1(h1, degparts.reshape(NC, NRB, RB, H))
    acc1 = _sc_spmm(g1.reshape(NC * N, H), src_all, dst_t, zeros_h, H)
    g2 = _tc_mid(acc1, g1, dinv, b1.reshape(1, D), W2)
    acc2 = _sc_spmm_edgesplit(g2, src_w, dst_w2, zeros_do, DO)
    return _tc_final(acc2, g2, dinv, b2.reshape(1, DO))


# speedup vs baseline: 18.3804x; 1.1350x over previous
import functools

import jax
import jax.numpy as jnp
from jax import lax
from jax.experimental import pallas as pl
from jax.experimental.pallas import tpu as pltpu
from jax.experimental.pallas import tpu_sc as plsc

N = 10000
E = 160000
NS = 16
NC = 2
WRITERS = 10
ROWS_PER_WRITER = N // WRITERS
EDGES_PER_TILE = E // NS
CHUNK = 100
CHUNKP = 104
NCHUNK = EDGES_PER_TILE // CHUNK
EDGES_PER_WORKER = E // (NC * NS)
CHUNK2 = 100
CHUNKP2 = 104
NCHUNK2 = EDGES_PER_WORKER // CHUNK2
DCHUNK = 125
DNCHUNK = EDGES_PER_WORKER // DCHUNK
RB = 1000
NRB = N // RB

_MESH = plsc.VectorSubcoreMesh(core_axis_name="c", subcore_axis_name="s")


def _sc_degree(dst_w, zeros_nf, ones_blk, W):
    @functools.partial(
        pl.kernel,
        out_type=jax.ShapeDtypeStruct((NC, N, W), jnp.float32),
        mesh=_MESH,
        scratch_types=[
            pltpu.VMEM((DNCHUNK, DCHUNK), jnp.int32),
            pltpu.VMEM((DCHUNK, W), jnp.float32),
            pltpu.VMEM_SHARED((N, W), jnp.float32),
            pltpu.SemaphoreType.DMA,
        ],
    )
    def k(dst_hbm, z_hbm, ones_hbm, out_hbm, dbuf, ones_v, accd, sem):
        c = lax.axis_index("c")
        s = lax.axis_index("s")
        pltpu.sync_copy(dst_hbm.at[c, s], dbuf)
        pltpu.sync_copy(ones_hbm, ones_v)
        r0 = s * ROWS_PER_WRITER

        @pl.when(s < WRITERS)
        def _():
            pltpu.sync_copy(z_hbm.at[pl.ds(r0, ROWS_PER_WRITER)],
                            accd.at[pl.ds(r0, ROWS_PER_WRITER)])

        plsc.subcore_barrier()

        @pl.loop(0, DNCHUNK)
        def _(j):
            pltpu.async_copy(ones_v, accd.at[dbuf.at[j]], sem, add=True)

        @pl.loop(0, DNCHUNK)
        def _(j):
            pltpu.make_async_copy(ones_v, accd.at[dbuf.at[j]], sem).wait()

        plsc.subcore_barrier()

        @pl.when(s < WRITERS)
        def _():
            pltpu.sync_copy(accd.at[pl.ds(r0, ROWS_PER_WRITER)],
                            out_hbm.at[c, pl.ds(r0, ROWS_PER_WRITER)])

    return k(dst_w, zeros_nf, ones_blk)


def _sc_spmm(g_flat, src_all, dst_t, zeros_nf, F):
    @functools.partial(
        pl.kernel,
        out_type=jax.ShapeDtypeStruct((NC, N, F), jnp.float32),
        mesh=_MESH,
        scratch_types=[
            pltpu.VMEM((NCHUNK * CHUNKP,), jnp.int32),
            pltpu.VMEM((NCHUNK, CHUNK), jnp.int32),
            pltpu.VMEM((CHUNK, F), jnp.float32),
            pltpu.VMEM((CHUNK, F), jnp.float32),
            pltpu.VMEM_SHARED((N, F), jnp.float32),
            pltpu.SemaphoreType.DMA,
            pltpu.SemaphoreType.DMA,
            pltpu.SemaphoreType.DMA,
            pltpu.SemaphoreType.DMA,
        ],
    )
    def k(g_hbm, src_hbm, dst_hbm, z_hbm, out_hbm,
          sbuf, dbuf, mb0, mb1, acc, sem0, sem1, ssem0, ssem1):
        c = lax.axis_index("c")
        s = lax.axis_index("s")
        pltpu.sync_copy(src_hbm.at[c, s], sbuf)
        pltpu.sync_copy(dst_hbm.at[s], dbuf)
        r0 = s * ROWS_PER_WRITER

        @pl.when(s < WRITERS)
        def _():
            pltpu.sync_copy(z_hbm.at[pl.ds(r0, ROWS_PER_WRITER)],
                            acc.at[pl.ds(r0, ROWS_PER_WRITER)])

        plsc.subcore_barrier()

        def sidx(j):
            return sbuf.at[pl.ds(pl.multiple_of(j * CHUNKP, 8), CHUNK)]

        pltpu.async_copy(g_hbm.at[sidx(0)], mb0, sem0)
        pltpu.async_copy(g_hbm.at[sidx(1)], mb1, sem1)

        @pl.loop(0, NCHUNK, step=2)
        def _(j):
            pltpu.make_async_copy(g_hbm.at[sidx(j)], mb0, sem0).wait()
            pltpu.async_copy(mb0, acc.at[dbuf.at[j]], ssem0, add=True)
            pltpu.make_async_copy(g_hbm.at[sidx(j + 1)], mb1, sem1).wait()
            pltpu.make_async_copy(mb0, acc.at[dbuf.at[j]], ssem0).wait()

            @pl.when(j + 2 < NCHUNK)
            def _():
                pltpu.async_copy(g_hbm.at[sidx(j + 2)], mb0, sem0)

            pltpu.async_copy(mb1, acc.at[dbuf.at[j + 1]], ssem1, add=True)
            pltpu.make_async_copy(mb1, acc.at[dbuf.at[j + 1]], ssem1).wait()

            @pl.when(j + 3 < NCHUNK)
            def _():
                pltpu.async_copy(g_hbm.at[sidx(j + 3)], mb1, sem1)

        plsc.subcore_barrier()

        @pl.when(s < WRITERS)
        def _():
            pltpu.sync_copy(acc.at[pl.ds(r0, ROWS_PER_WRITER)],
                            out_hbm.at[c, pl.ds(r0, ROWS_PER_WRITER)])

    return k(g_flat, src_all, dst_t, zeros_nf)


def _sc_spmm_edgesplit(g, src_w, dst_w2, zeros_nf, F):
    @functools.partial(
        pl.kernel,
        out_type=jax.ShapeDtypeStruct((NC, N, F), jnp.float32),
        mesh=_MESH,
        scratch_types=[
            pltpu.VMEM((NCHUNK2 * CHUNKP2,), jnp.int32),
            pltpu.VMEM((NCHUNK2, CHUNK2), jnp.int32),
            pltpu.VMEM((CHUNK2, F), jnp.float32),
            pltpu.VMEM((CHUNK2, F), jnp.float32),
            pltpu.VMEM_SHARED((N, F), jnp.float32),
            pltpu.SemaphoreType.DMA,
            pltpu.SemaphoreType.DMA,
            pltpu.SemaphoreType.DMA,
            pltpu.SemaphoreType.DMA,
        ],
    )
    def k(g_hbm, src_hbm, dst_hbm, z_hbm, out_hbm,
          sbuf, dbuf, mb0, mb1, acc, sem0, sem1, ssem0, ssem1):
        c = lax.axis_index("c")
        s = lax.axis_index("s")
        pltpu.sync_copy(src_hbm.at[c, s], sbuf)
        pltpu.sync_copy(dst_hbm.at[c, s], dbuf)
        r0 = s * ROWS_PER_WRITER

        @pl.when(s < WRITERS)
        def _():
            pltpu.sync_copy(z_hbm.at[pl.ds(r0, ROWS_PER_WRITER)],
                            acc.at[pl.ds(r0, ROWS_PER_WRITER)])

        plsc.subcore_barrier()

        def sidx(j):
            return sbuf.at[pl.ds(pl.multiple_of(j * CHUNKP2, 8), CHUNK2)]

        pltpu.async_copy(g_hbm.at[sidx(0)], mb0, sem0)
        pltpu.async_copy(g_hbm.at[sidx(1)], mb1, sem1)

        @pl.loop(0, NCHUNK2, step=2)
        def _(j):
            pltpu.make_async_copy(g_hbm.at[sidx(j)], mb0, sem0).wait()
            pltpu.async_copy(mb0, acc.at[dbuf.at[j]], ssem0, add=True)
            pltpu.make_async_copy(g_hbm.at[sidx(j + 1)], mb1, sem1).wait()
            pltpu.make_async_copy(mb0, acc.at[dbuf.at[j]], ssem0).wait()

            @pl.when(j + 2 < NCHUNK2)
            def _():
                pltpu.async_copy(g_hbm.at[sidx(j + 2)], mb0, sem0)

            pltpu.async_copy(mb1, acc.at[dbuf.at[j + 1]], ssem1, add=True)
            pltpu.make_async_copy(mb1, acc.at[dbuf.at[j + 1]], ssem1).wait()

            @pl.when(j + 3 < NCHUNK2)
            def _():
                pltpu.async_copy(g_hbm.at[sidx(j + 3)], mb1, sem1)

        plsc.subcore_barrier()

        @pl.when(s < WRITERS)
        def _():
            pltpu.sync_copy(acc.at[pl.ds(r0, ROWS_PER_WRITER)],
                            out_hbm.at[c, pl.ds(r0, ROWS_PER_WRITER)])

    return k(g, src_w, dst_w2, zeros_nf)


def _tc_mm1(x, W1):
    D = W1.shape[0]

    def body(x_ref, w_ref, o_ref):
        o_ref[...] = jnp.dot(x_ref[...], w_ref[...],
                             preferred_element_type=jnp.float32)

    return pl.pallas_call(
        body,
        grid=(NRB,),
        in_specs=[pl.BlockSpec((RB, D), lambda i: (i, 0)),
                  pl.BlockSpec((D, D), lambda i: (0, 0))],
        out_specs=pl.BlockSpec((RB, D), lambda i: (i, 0)),
        out_shape=jax.ShapeDtypeStruct((N, D), jnp.float32),
    )(x, W1)


def _tc_scale1(h1, degparts):
    D = h1.shape[1]
    H = D // 2

    def body(h_ref, d_ref, g_ref, dinv_ref):
        deg = d_ref[0, 0, :, 0:1] + d_ref[1, 0, :, 0:1] + 1.0
        dinv = lax.rsqrt(deg)
        dinv_ref[0] = dinv
        g = h_ref[...] * dinv
        g_ref[0] = g[:, :H]
        g_ref[1] = g[:, H:]

    return pl.pallas_call(
        body,
        grid=(NRB,),
        in_specs=[pl.BlockSpec((RB, D), lambda i: (i, 0)),
                  pl.BlockSpec((NC, 1, RB, H), lambda i: (0, i, 0, 0))],
        out_specs=[pl.BlockSpec((NC, RB, H), lambda i: (0, i, 0)),
                   pl.BlockSpec((1, RB, 1), lambda i: (i, 0, 0))],
        out_shape=[jax.ShapeDtypeStruct((NC, N, H), jnp.float32),
                   jax.ShapeDtypeStruct((NRB, RB, 1), jnp.float32)],
    )(h1, degparts)


def _tc_mid(acc1, g1, dinv, b1, W2):
    H = acc1.shape[2]
    D = 2 * H
    DO = W2.shape[1]

    def body(a_ref, g_ref, d_ref, b_ref, w_ref, o_ref):
        dv = d_ref[0]
        full = jnp.concatenate([a_ref[0] + g_ref[0], a_ref[1] + g_ref[1]],
                               axis=1)
        h = jnp.maximum(full * dv + b_ref[...], 0.0)
        hw = jnp.dot(h, w_ref[...], preferred_element_type=jnp.float32)
        o_ref[...] = hw * dv

    return pl.pallas_call(
        body,
        grid=(NRB,),
        in_specs=[pl.BlockSpec((NC, RB, H), lambda i: (0, i, 0)),
                  pl.BlockSpec((NC, RB, H), lambda i: (0, i, 0)),
                  pl.BlockSpec((1, RB, 1), lambda i: (i, 0, 0)),
                  pl.BlockSpec((1, D), lambda i: (0, 0)),
                  pl.BlockSpec((D, DO), lambda i: (0, 0))],
        out_specs=pl.BlockSpec((RB, DO), lambda i: (i, 0)),
        out_shape=jax.ShapeDtypeStruct((N, DO), jnp.float32),
    )(acc1, g1, dinv, b1, W2)


def _tc_final(acc2, g2, dinv, b2):
    DO = g2.shape[1]

    def body(a_ref, g_ref, d_ref, b_ref, o_ref):
        dv = d_ref[0]
        o = (a_ref[0] + a_ref[1] + g_ref[...]) * dv + b_ref[...]
        m = jnp.max(o, axis=1, keepdims=True)
        e = jnp.exp(o - m)
        o_ref[...] = (o - m) - jnp.log(jnp.sum(e, axis=1, keepdims=True))

    return pl.pallas_call(
        body,
        grid=(NRB,),
        in_specs=[pl.BlockSpec((NC, RB, DO), lambda i: (0, i, 0)),
                  pl.BlockSpec((RB, DO), lambda i: (i, 0)),
                  pl.BlockSpec((1, RB, 1), lambda i: (i, 0, 0)),
                  pl.BlockSpec((1, DO), lambda i: (0, 0))],
        out_specs=pl.BlockSpec((RB, DO), lambda i: (i, 0)),
        out_shape=jax.ShapeDtypeStruct((N, DO), jnp.float32),
    )(acc2, g2, dinv, b2)


def kernel(x, edge_index, W1, b1, W2, b2):
    D = W1.shape[0]
    H = D // 2
    DO = W2.shape[1]
    src = edge_index[0]
    dst = edge_index[1]

    srcr = jnp.pad(src.reshape(NS, NCHUNK, CHUNK),
                   ((0, 0), (0, 0), (0, CHUNKP - CHUNK)))
    src_all = (srcr[None] + (jnp.arange(NC, dtype=jnp.int32) * N)[:, None, None, None]
               ).reshape(NC, NS, NCHUNK * CHUNKP)
    dst_t = dst.reshape(NS, NCHUNK, CHUNK)
    src_w = jnp.pad(src.reshape(NC, NS, NCHUNK2, CHUNK2),
                    ((0, 0), (0, 0), (0, 0), (0, CHUNKP2 - CHUNK2))
                    ).reshape(NC, NS, NCHUNK2 * CHUNKP2)
    dst_w2 = dst.reshape(NC, NS, NCHUNK2, CHUNK2)
    dst_w = dst.reshape(NC, NS, DNCHUNK, DCHUNK)
    ones_blk = jnp.ones((DCHUNK, H), jnp.float32)
    zeros_h = jnp.zeros((N, H), jnp.float32)
    zeros_do = jnp.zeros((N, DO), jnp.float32)

    degparts = _sc_degree(dst_w, zeros_h, ones_blk, H)
    h1 = _tc_mm1(x, W1)
    g1, dinv = _tc_scale1(h1, degparts.reshape(NC, NRB, RB, H))
    acc1 = _sc_spmm(g1.reshape(NC * N, H), src_all, dst_t, zeros_h, H)
    g2 = _tc_mid(acc1, g1, dinv, b1.reshape(1, D), W2)
    acc2 = _sc_spmm_edgesplit(g2, src_w, dst_w2, zeros_do, DO)
    return _tc_final(acc2, g2, dinv, b2.reshape(1, DO))
